# baseline (speedup 1.0000x reference)
import functools

import jax
import jax.numpy as jnp
from jax import lax
from jax.experimental import pallas as pl
from jax.experimental.pallas import tpu as pltpu
from jax.experimental.pallas import tpu_sc as plsc

B = 16384
D = 64
NC = 2
NS = 16
L = 16
W = 256
N_BIG = 1_000_000
N_SMALL = 100_000
SUB = 256
BPW2 = B // NS

SCR_BIG = 500_096
SCR_SMALL = 50_048


def _transpose_table(tT, scr, n_rows, sid, slab_v, rowbuf, half_iota, colbase):
    nfull = n_rows // W
    per = (nfull + 1 + NS - 1) // NS
    tail_c0 = -(-n_rows // 128) * 128 - W

    def t_body(t, _):
        s = sid + NS * t

        @pl.when(s <= nfull)
        def _():
            c0 = pl.multiple_of(
                jnp.where(s == nfull, jnp.int32(tail_c0), s * W), 128)
            pltpu.sync_copy(tT.at[:, pl.ds(c0, W)], slab_v)

            def g_body(g, _):
                pvec = jax.lax.full((L,), g * 8, jnp.int32) + half_iota
                gl = g * L
                for j in range(D):
                    x = slab_v[j, pl.ds(gl, L)]
                    plsc.store_scatter(rowbuf, [pvec, colbase + j], x)
                return ()

            jax.lax.fori_loop(0, W // L, g_body, (), unroll=False)
            r0 = pl.multiple_of(c0 // 2, 64)
            pltpu.sync_copy(rowbuf, scr.at[pl.ds(r0, W // 2)])
        return ()

    jax.lax.fori_loop(0, per, t_body, (), unroll=False)


def _gather_set(idx_hbm, scr, outT, sid, idx_v, q_v, rows_v, outT_v, sem,
                iota):
    def sb_body(sb, _):
        base = pl.multiple_of(sid * BPW2 + sb * SUB, SUB)
        pltpu.sync_copy(idx_hbm.at[pl.ds(base, SUB)], idx_v)

        def q_body(g, _):
            v = idx_v[pl.ds(g * L, L)]
            q_v[pl.ds(g * L, L)] = jax.lax.shift_right_logical(v, 1)
            return ()

        jax.lax.fori_loop(0, SUB // L, q_body, (), unroll=False)
        descs = []
        for c in range(SUB // 128):
            descs.append(pltpu.async_copy(
                scr.at[q_v.at[pl.ds(c * 128, 128)]],
                rows_v.at[pl.ds(c * 128, 128)], sem))
        for d in descs:
            d.wait()

        def sel_body(g, _):
            h64 = jax.lax.shift_left(
                jax.lax.bitwise_and(idx_v[pl.ds(g * L, L)], 1), 6)
            bvec = jax.lax.full((L,), g * L, jnp.int32) + iota

            def j_body(j, _):
                outT_v[j, pl.ds(g * L, L)] = plsc.load_gather(
                    rows_v, [bvec, h64 + j])
                return ()

            jax.lax.fori_loop(0, D, j_body, (), unroll=False)
            return ()

        jax.lax.fori_loop(0, SUB // L, sel_body, (), unroll=False)
        pltpu.sync_copy(outT_v, outT.at[:, pl.ds(base, SUB)])
        return ()

    jax.lax.fori_loop(0, BPW2 // SUB, sb_body, (), unroll=False)


def _sc_body(aT, dT, vT, fT,
             idx_user, idx_wrote, idx_cited, idx_coauthor, idx_venue,
             idx_affil,
             o_user, o_wrote, o_cited, o_coauthor, o_venue, o_affil,
             scA, scD, scV, scF,
             slab_v, rowbuf, idx_v, q_v, rows_v, outT_v, sem):
    core = lax.axis_index("c")
    sid = lax.axis_index("s")
    iota = jax.lax.iota(jnp.int32, L)
    half_iota = jax.lax.shift_right_logical(iota, 1)
    colbase = jax.lax.shift_left(jax.lax.bitwise_and(iota, 1), 6)

    @pl.when(core == 0)
    def _():
        _transpose_table(aT, scA, N_BIG, sid, slab_v, rowbuf, half_iota,
                         colbase)
        _transpose_table(vT, scV, N_SMALL, sid, slab_v, rowbuf, half_iota,
                         colbase)

    @pl.when(core == 1)
    def _():
        _transpose_table(dT, scD, N_BIG, sid, slab_v, rowbuf, half_iota,
                         colbase)
        _transpose_table(fT, scF, N_SMALL, sid, slab_v, rowbuf, half_iota,
                         colbase)

    plsc.subcore_barrier()

    @pl.when(core == 0)
    def _():
        for idx_hbm, scr, outT in ((idx_user, scA, o_user),
                                   (idx_coauthor, scA, o_coauthor),
                                   (idx_venue, scV, o_venue)):
            _gather_set(idx_hbm, scr, outT, sid, idx_v, q_v, rows_v, outT_v,
                        sem, iota)

    @pl.when(core == 1)
    def _():
        for idx_hbm, scr, outT in ((idx_wrote, scD, o_wrote),
                                   (idx_cited, scD, o_cited),
                                   (idx_affil, scF, o_affil)):
            _gather_set(idx_hbm, scr, outT, sid, idx_v, q_v, rows_v, outT_v,
                        sem, iota)


@functools.cache
def _make_sc():
    return pl.kernel(
        _sc_body,
        mesh=plsc.VectorSubcoreMesh(core_axis_name="c", subcore_axis_name="s"),
        out_type=[jax.ShapeDtypeStruct((D, B), jnp.float32)] * 6 + [
            jax.ShapeDtypeStruct((SCR_BIG, 128), jnp.float32),
            jax.ShapeDtypeStruct((SCR_BIG, 128), jnp.float32),
            jax.ShapeDtypeStruct((SCR_SMALL, 128), jnp.float32),
            jax.ShapeDtypeStruct((SCR_SMALL, 128), jnp.float32),
        ],
        scratch_types=[
            pltpu.VMEM((D, W), jnp.float32),
            pltpu.VMEM((W // 2, 128), jnp.float32),
            pltpu.VMEM((SUB,), jnp.int32),
            pltpu.VMEM((SUB,), jnp.int32),
            pltpu.VMEM((SUB, 128), jnp.float32),
            pltpu.VMEM((D, SUB), jnp.float32),
            pltpu.SemaphoreType.DMA,
        ],
        compiler_params=pltpu.CompilerParams(needs_layout_passes=False),
    )


_TC_BLOCK = 2048


def _tc_bcast_body(relT_ref, o0, o1, o2, o3, o4):
    relT = relT_ref[...]
    for k, o in enumerate((o0, o1, o2, o3, o4)):
        o[...] = jnp.broadcast_to(relT[:, k:k + 1], (D, _TC_BLOCK))


def _tc_bcast(relation_table):
    relT = jnp.zeros((D, 8), jnp.float32).at[:, :5].set(relation_table.T)
    return pl.pallas_call(
        _tc_bcast_body,
        grid=(B // _TC_BLOCK,),
        in_specs=[pl.BlockSpec((D, 8), lambda i: (0, 0))],
        out_specs=[pl.BlockSpec((D, _TC_BLOCK), lambda i: (0, i))] * 5,
        out_shape=[jax.ShapeDtypeStruct((D, B), jnp.float32)] * 5,
    )(relT)


def kernel(user_id, wrote, cited, coauthor, venue, affiliation,
           author_table, venue_table, affiliation_table, relation_table,
           doc_embs):
    def prep(ix):
        return ix.astype(jnp.int32)

    outs = _make_sc()(
        author_table.T, doc_embs.T, venue_table.T, affiliation_table.T,
        prep(user_id), prep(wrote), prep(cited), prep(coauthor),
        prep(venue), prep(affiliation))
    user_e, wrote_e, cited_e, coauthor_e, venue_e, affil_e = (
        o.T for o in outs[:6])
    rel_outs = _tc_bcast(relation_table)
    wrote_r, cited_r, coauth_r, venue_r, affil_r = (o.T for o in rel_outs)
    return (user_e, wrote_e, cited_e, coauthor_e, venue_e, affil_e,
            wrote_r, cited_r, coauth_r, venue_r, affil_r)


# speedup vs baseline: 1.1378x; 1.1378x over previous
import functools

import jax
import jax.numpy as jnp
from jax import lax
from jax.experimental import pallas as pl
from jax.experimental.pallas import tpu as pltpu
from jax.experimental.pallas import tpu_sc as plsc

B = 16384
D = 64
NC = 2
NS = 16
L = 16
W = 256
N_BIG = 1_000_000
N_SMALL = 100_000
SUB = 256
BPW2 = B // NS

SCR_BIG = 500_096
SCR_SMALL = 50_048


def _transpose_table(tT, scr, n_rows, sid, slab_v, rowbuf, half_iota, colbase):
    nfull = n_rows // W
    per = (nfull + 1 + NS - 1) // NS
    tail_c0 = -(-n_rows // 128) * 128 - W

    def t_body(t, _):
        s = sid + NS * t

        @pl.when(s <= nfull)
        def _():
            c0 = pl.multiple_of(
                jnp.where(s == nfull, jnp.int32(tail_c0), s * W), 128)
            pltpu.sync_copy(tT.at[:, pl.ds(c0, W)], slab_v)

            @plsc.parallel_loop(0, W // L)
            def g_body(g):
                pvec = jax.lax.full((L,), g * 8, jnp.int32) + half_iota
                gl = pl.multiple_of(g * L, L)
                for j in range(D):
                    x = slab_v[j, pl.ds(gl, L)]
                    plsc.store_scatter(rowbuf, [pvec, colbase + j], x)

            r0 = pl.multiple_of(c0 // 2, 64)
            pltpu.sync_copy(rowbuf, scr.at[pl.ds(r0, W // 2)])
        return ()

    jax.lax.fori_loop(0, per, t_body, (), unroll=False)


def _gather_set(idx_hbm, scr, outT, sid, idx_v, q_v, rows_v, outT_v, sem,
                iota):
    def sb_body(sb, _):
        base = pl.multiple_of(sid * BPW2 + sb * SUB, SUB)
        pltpu.sync_copy(idx_hbm.at[pl.ds(base, SUB)], idx_v)

        def q_body(g, _):
            gl = pl.multiple_of(g * L, L)
            v = idx_v[pl.ds(gl, L)]
            q_v[pl.ds(gl, L)] = jax.lax.shift_right_logical(v, 1)
            return ()

        jax.lax.fori_loop(0, SUB // L, q_body, (), unroll=False)

        descs = []
        for c in range(SUB // 128):
            descs.append(pltpu.async_copy(
                scr.at[q_v.at[pl.ds(c * 128, 128)]],
                rows_v.at[pl.ds(c * 128, 128)], sem))
        for d in descs:
            d.wait()

        def sel_body(g, _):
            gl = pl.multiple_of(g * L, L)
            h64 = jax.lax.shift_left(
                jax.lax.bitwise_and(idx_v[pl.ds(gl, L)], 1), 6)
            bvec = jax.lax.full((L,), gl, jnp.int32) + iota

            def j_body(j, _):
                outT_v[j, pl.ds(gl, L)] = plsc.load_gather(
                    rows_v, [bvec, h64 + j])
                return ()

            jax.lax.fori_loop(0, D, j_body, (), unroll=False)
            return ()

        jax.lax.fori_loop(0, SUB // L, sel_body, (), unroll=False)

        pltpu.sync_copy(outT_v, outT.at[:, pl.ds(base, SUB)])
        return ()

    jax.lax.fori_loop(0, BPW2 // SUB, sb_body, (), unroll=False)


def _sc_body(aT, dT, vT, fT,
             idx_user, idx_wrote, idx_cited, idx_coauthor, idx_venue,
             idx_affil,
             o_user, o_wrote, o_cited, o_coauthor, o_venue, o_affil,
             scA, scD, scV, scF,
             slab_v, rowbuf, idx_v, q_v, rows_v, outT_v, sem):
    core = lax.axis_index("c")
    sid = lax.axis_index("s")
    iota = jax.lax.iota(jnp.int32, L)
    half_iota = jax.lax.shift_right_logical(iota, 1)
    colbase = jax.lax.shift_left(jax.lax.bitwise_and(iota, 1), 6)

    @pl.when(core == 0)
    def _():
        _transpose_table(aT, scA, N_BIG, sid, slab_v, rowbuf, half_iota,
                         colbase)
        _transpose_table(vT, scV, N_SMALL, sid, slab_v, rowbuf, half_iota,
                         colbase)

    @pl.when(core == 1)
    def _():
        _transpose_table(dT, scD, N_BIG, sid, slab_v, rowbuf, half_iota,
                         colbase)
        _transpose_table(fT, scF, N_SMALL, sid, slab_v, rowbuf, half_iota,
                         colbase)

    plsc.subcore_barrier()

    @pl.when(core == 0)
    def _():
        for idx_hbm, scr, outT in ((idx_user, scA, o_user),
                                   (idx_coauthor, scA, o_coauthor),
                                   (idx_venue, scV, o_venue)):
            _gather_set(idx_hbm, scr, outT, sid, idx_v, q_v, rows_v, outT_v,
                        sem, iota)

    @pl.when(core == 1)
    def _():
        for idx_hbm, scr, outT in ((idx_wrote, scD, o_wrote),
                                   (idx_cited, scD, o_cited),
                                   (idx_affil, scF, o_affil)):
            _gather_set(idx_hbm, scr, outT, sid, idx_v, q_v, rows_v, outT_v,
                        sem, iota)


@functools.cache
def _make_sc():
    return pl.kernel(
        _sc_body,
        mesh=plsc.VectorSubcoreMesh(core_axis_name="c", subcore_axis_name="s"),
        out_type=[jax.ShapeDtypeStruct((D, B), jnp.float32)] * 6 + [
            jax.ShapeDtypeStruct((SCR_BIG, 128), jnp.float32),
            jax.ShapeDtypeStruct((SCR_BIG, 128), jnp.float32),
            jax.ShapeDtypeStruct((SCR_SMALL, 128), jnp.float32),
            jax.ShapeDtypeStruct((SCR_SMALL, 128), jnp.float32),
        ],
        scratch_types=[
            pltpu.VMEM((D, W), jnp.float32),
            pltpu.VMEM((W // 2, 128), jnp.float32),
            pltpu.VMEM((SUB,), jnp.int32),
            pltpu.VMEM((SUB,), jnp.int32),
            pltpu.VMEM((SUB, 128), jnp.float32),
            pltpu.VMEM((D, SUB), jnp.float32),
            pltpu.SemaphoreType.DMA,
        ],
        compiler_params=pltpu.CompilerParams(needs_layout_passes=False),
    )


_TC_BLOCK = 2048


def _tc_bcast_body(relT_ref, o0, o1, o2, o3, o4):
    relT = relT_ref[...]
    for k, o in enumerate((o0, o1, o2, o3, o4)):
        o[...] = jnp.broadcast_to(relT[:, k:k + 1], (D, _TC_BLOCK))


def _tc_bcast(relation_table):
    relT = jnp.zeros((D, 8), jnp.float32).at[:, :5].set(relation_table.T)
    return pl.pallas_call(
        _tc_bcast_body,
        grid=(B // _TC_BLOCK,),
        in_specs=[pl.BlockSpec((D, 8), lambda i: (0, 0))],
        out_specs=[pl.BlockSpec((D, _TC_BLOCK), lambda i: (0, i))] * 5,
        out_shape=[jax.ShapeDtypeStruct((D, B), jnp.float32)] * 5,
    )(relT)


def kernel(user_id, wrote, cited, coauthor, venue, affiliation,
           author_table, venue_table, affiliation_table, relation_table,
           doc_embs):
    def prep(ix):
        return ix.astype(jnp.int32)

    outs = _make_sc()(
        author_table.T, doc_embs.T, venue_table.T, affiliation_table.T,
        prep(user_id), prep(wrote), prep(cited), prep(coauthor),
        prep(venue), prep(affiliation))
    user_e, wrote_e, cited_e, coauthor_e, venue_e, affil_e = (
        o.T for o in outs[:6])
    rel_outs = _tc_bcast(relation_table)
    wrote_r, cited_r, coauth_r, venue_r, affil_r = (o.T for o in rel_outs)
    return (user_e, wrote_e, cited_e, coauthor_e, venue_e, affil_e,
            wrote_r, cited_r, coauth_r, venue_r, affil_r)


# speedup vs baseline: 2.6295x; 2.3110x over previous
import functools

import jax
import jax.numpy as jnp
from jax import lax
from jax.experimental import pallas as pl
from jax.experimental.pallas import tpu as pltpu
from jax.experimental.pallas import tpu_sc as plsc

B = 16384
D = 64
NC = 2
NS = 16
NW = NC * NS
BPW = B // NW
CHUNK = 128
NCH = BPW // CHUNK


def _sc_gather_body(idx0, idx1, idx2, idx3, idx4, idx5,
                    author_t, doc_t, venue_t, affil_t,
                    out0, out1, out2, out3, out4, out5,
                    idx_v, rows_v, sem):
    wid = lax.axis_index("s") * NC + lax.axis_index("c")
    row0 = wid * NCH

    jobs = ((idx0, author_t, out0),
            (idx1, doc_t, out1),
            (idx2, doc_t, out2),
            (idx3, author_t, out3),
            (idx4, venue_t, out4),
            (idx5, affil_t, out5))

    for idx_hbm, table_hbm, out_hbm in jobs:
        pltpu.sync_copy(idx_hbm.at[pl.ds(row0, NCH)], idx_v)
        descs = []
        for j in range(NCH):
            descs.append(pltpu.async_copy(
                table_hbm.at[idx_v.at[j]],
                rows_v.at[pl.ds(j * CHUNK, CHUNK)],
                sem))
        for dsc in descs:
            dsc.wait()
        pltpu.sync_copy(rows_v, out_hbm.at[pl.ds(wid * BPW, BPW)])


@functools.cache
def _make_sc_gather():
    return pl.kernel(
        _sc_gather_body,
        mesh=plsc.VectorSubcoreMesh(core_axis_name="c", subcore_axis_name="s"),
        out_type=[jax.ShapeDtypeStruct((B, D), jnp.float32)] * 6,
        scratch_types=[
            pltpu.VMEM((NCH, CHUNK), jnp.int32),
            pltpu.VMEM((BPW, D), jnp.float32),
            pltpu.SemaphoreType.DMA,
        ],
        compiler_params=pltpu.CompilerParams(use_tc_tiling_on_sc=False),
    )


_TC_BLOCK = 1024


def _tc_bcast_body(rel_ref, o0, o1, o2, o3, o4):
    rel = rel_ref[...]
    for k, o in enumerate((o0, o1, o2, o3, o4)):
        o[...] = jnp.broadcast_to(rel[k][None, :], (_TC_BLOCK, D))


def _tc_bcast(relation_table):
    return pl.pallas_call(
        _tc_bcast_body,
        grid=(B // _TC_BLOCK,),
        in_specs=[pl.BlockSpec((5, D), lambda i: (0, 0))],
        out_specs=[pl.BlockSpec((_TC_BLOCK, D), lambda i: (i, 0))] * 5,
        out_shape=[jax.ShapeDtypeStruct((B, D), jnp.float32)] * 5,
    )(relation_table)


def kernel(user_id, wrote, cited, coauthor, venue, affiliation,
           author_table, venue_table, affiliation_table, relation_table,
           doc_embs):
    def prep(ix):
        return ix.astype(jnp.int32).reshape(B // CHUNK, CHUNK)

    outs = _make_sc_gather()(
        prep(user_id), prep(wrote), prep(cited), prep(coauthor),
        prep(venue), prep(affiliation),
        author_table, doc_embs, venue_table, affiliation_table)
    user_e, wrote_e, cited_e, coauthor_e, venue_e, affil_e = outs
    wrote_r, cited_r, coauth_r, venue_r, affil_r = _tc_bcast(relation_table)
    return (user_e, wrote_e, cited_e, coauthor_e, venue_e, affil_e,
            wrote_r, cited_r, coauth_r, venue_r, affil_r)


# speedup vs baseline: 2.7091x; 1.0303x over previous
import functools

import jax
import jax.numpy as jnp
from jax import lax
from jax.experimental import pallas as pl
from jax.experimental.pallas import tpu as pltpu
from jax.experimental.pallas import tpu_sc as plsc

B = 16384
D = 64
NC = 2
NS = 16
NW = NC * NS
BPW = B // NW
CHUNK = 128
NCH = BPW // CHUNK


def _sc_gather_body(idx0, idx1, idx2, idx3, idx4, idx5,
                    author_t, doc_t, venue_t, affil_t,
                    out0, out1, out2, out3, out4, out5,
                    idx_v, rows_v, sem):
    wid = lax.axis_index("s") * NC + lax.axis_index("c")
    row0 = wid * NCH

    jobs = ((idx0, author_t, out0),
            (idx1, doc_t, out1),
            (idx2, doc_t, out2),
            (idx3, author_t, out3),
            (idx4, venue_t, out4),
            (idx5, affil_t, out5))

    for idx_hbm, table_hbm, out_hbm in jobs:
        pltpu.sync_copy(idx_hbm.at[pl.ds(row0, NCH)], idx_v)
        descs = []
        for j in range(NCH):
            descs.append(pltpu.async_copy(
                table_hbm.at[idx_v.at[j]],
                rows_v.at[pl.ds(j * CHUNK, CHUNK)],
                sem))
        for dsc in descs:
            dsc.wait()
        pltpu.sync_copy(rows_v, out_hbm.at[pl.ds(wid * BPW, BPW)])


@functools.cache
def _make_sc_gather():
    return pl.kernel(
        _sc_gather_body,
        mesh=plsc.VectorSubcoreMesh(core_axis_name="c", subcore_axis_name="s"),
        out_type=[jax.ShapeDtypeStruct((B, D), jnp.float32)] * 6,
        scratch_types=[
            pltpu.VMEM((NCH, CHUNK), jnp.int32),
            pltpu.VMEM((BPW, D), jnp.float32),
            pltpu.SemaphoreType.DMA,
        ],
        compiler_params=pltpu.CompilerParams(use_tc_tiling_on_sc=False),
    )


_TC_BLOCK = 2048


def _tc_bcast_body(relT_ref, o0, o1, o2, o3, o4):
    relT = relT_ref[...]
    for k, o in enumerate((o0, o1, o2, o3, o4)):
        o[...] = jnp.broadcast_to(relT[:, k:k + 1], (D, _TC_BLOCK))


def _tc_bcast(relation_table):
    relT = jnp.zeros((D, 8), jnp.float32).at[:, :5].set(relation_table.T)
    return pl.pallas_call(
        _tc_bcast_body,
        grid=(B // _TC_BLOCK,),
        in_specs=[pl.BlockSpec((D, 8), lambda i: (0, 0))],
        out_specs=[pl.BlockSpec((D, _TC_BLOCK), lambda i: (0, i))] * 5,
        out_shape=[jax.ShapeDtypeStruct((D, B), jnp.float32)] * 5,
    )(relT)


def kernel(user_id, wrote, cited, coauthor, venue, affiliation,
           author_table, venue_table, affiliation_table, relation_table,
           doc_embs):
    def prep(ix):
        return ix.astype(jnp.int32).reshape(B // CHUNK, CHUNK)

    outs = _make_sc_gather()(
        prep(user_id), prep(wrote), prep(cited), prep(coauthor),
        prep(venue), prep(affiliation),
        author_table, doc_embs, venue_table, affiliation_table)
    user_e, wrote_e, cited_e, coauthor_e, venue_e, affil_e = outs
    wrote_r, cited_r, coauth_r, venue_r, affil_r = (
        o.T for o in _tc_bcast(relation_table))
    return (user_e, wrote_e, cited_e, coauthor_e, venue_e, affil_e,
            wrote_r, cited_r, coauth_r, venue_r, affil_r)
